# baseline (speedup 1.0000x reference)
import functools

import jax
import jax.numpy as jnp
from jax import lax
from jax.experimental import pallas as pl
from jax.experimental.pallas import tpu as pltpu
from jax.experimental.pallas import tpu_sc as plsc

N_TOKENS = 4608
D = 768
K = 1024
TB = 512
CHUNKS = (2048, 1536, 1024)
OFFS = (0, 2048, 3584)


def _argmin_body(e_ref, x_ref, x2_ref, e2_ref, idx_ref, et_ref):
    @pl.when(pl.program_id(0) == 0)
    def _():
        et_ref[...] = jnp.transpose(e_ref[...], (1, 0))

    scores = lax.dot_general(
        x_ref[...].astype(jnp.bfloat16), et_ref[...].astype(jnp.bfloat16),
        (((1,), (0,)), ((), ())),
        preferred_element_type=jnp.float32,
    )
    d = (x2_ref[...] - 2.0 * scores) + e2_ref[...]
    m = jnp.min(d, axis=1, keepdims=True)
    col = lax.broadcasted_iota(jnp.int32, d.shape, 1)
    idx = jnp.min(jnp.where(d == m, col, K), axis=1)
    idx_ref[...] = idx.astype(jnp.int32)


def _argmin_indices(x_flat, x2, e2, embedding, off, ch):
    blk0 = off // TB
    out = pl.pallas_call(
        _argmin_body,
        grid=(ch // TB,),
        in_specs=[
            pl.BlockSpec((K, D), lambda i: (0, 0)),
            pl.BlockSpec((TB, D), lambda i, b=blk0: (b + i, 0)),
            pl.BlockSpec((TB, 1), lambda i, b=blk0: (b + i, 0)),
            pl.BlockSpec((1, K), lambda i: (0, 0)),
        ],
        out_specs=pl.BlockSpec((TB,), lambda i: (i,)),
        out_shape=jax.ShapeDtypeStruct((ch,), jnp.int32),
        scratch_shapes=[
            pltpu.VMEM((D, K), jnp.float32),
        ],
    )(embedding, x_flat, x2, e2)
    return out


def _make_gather(ch):
    info = plsc.get_sparse_core_info()
    nc, ns = info.num_cores, info.num_subcores
    nw = nc * ns
    b_per_w = ch // nw
    half = b_per_w // 2
    mesh = plsc.VectorSubcoreMesh(core_axis_name="c", subcore_axis_name="s")

    @functools.partial(
        pl.kernel,
        mesh=mesh,
        out_type=jax.ShapeDtypeStruct((ch, D), jnp.float32),
        scratch_types=[
            pltpu.VMEM((b_per_w,), jnp.int32),
            pltpu.VMEM((half, D), jnp.float32),
            pltpu.VMEM((half, D), jnp.float32),
            pltpu.SemaphoreType.DMA,
            pltpu.SemaphoreType.DMA,
            pltpu.SemaphoreType.DMA,
            pltpu.SemaphoreType.DMA,
        ],
    )
    def gather(table_hbm, idx_hbm, out_hbm, idx_v, buf0, buf1, sg0, sg1, sw0, sw1):
        wid = lax.axis_index("s") * nc + lax.axis_index("c")
        base = wid * b_per_w
        pltpu.sync_copy(idx_hbm.at[pl.ds(base, b_per_w)], idx_v)
        g0 = pltpu.async_copy(table_hbm.at[idx_v.at[pl.ds(0, half)]], buf0, sg0)
        g1 = pltpu.async_copy(table_hbm.at[idx_v.at[pl.ds(half, half)]], buf1, sg1)
        g0.wait()
        w0 = pltpu.async_copy(buf0, out_hbm.at[pl.ds(base, half)], sw0)
        g1.wait()
        w1 = pltpu.async_copy(buf1, out_hbm.at[pl.ds(base + half, half)], sw1)
        w0.wait()
        w1.wait()

    return gather


def kernel(x, embedding):
    B, T, _ = x.shape
    x_flat = x.reshape(B * T, D)
    x2 = jnp.sum(x_flat ** 2, axis=1, keepdims=True)
    e2 = jnp.sum(embedding ** 2, axis=1)[None, :]
    quantized = jnp.empty((N_TOKENS, D), jnp.float32)
    for off, ch in zip(OFFS, CHUNKS):
        idx_c = _argmin_indices(x_flat, x2, e2, embedding, off, ch)
        q_c = _make_gather(ch)(embedding, idx_c)
        quantized = lax.dynamic_update_slice(quantized, q_c, (off, 0))
    return quantized.reshape(B, T, D)


# speedup vs baseline: 1.0524x; 1.0524x over previous
import functools

import jax
import jax.numpy as jnp
from jax import lax
from jax.experimental import pallas as pl
from jax.experimental.pallas import tpu as pltpu
from jax.experimental.pallas import tpu_sc as plsc

N_TOKENS = 4608
D = 768
K = 1024
TB = 512
CHUNKS = (2048, 1536, 1024)
OFFS = (0, 2048, 3584)


def _argmin_body(e_ref, x_ref, x2_ref, e2_ref, idx_ref, et_ref):
    @pl.when(pl.program_id(0) == 0)
    def _():
        et_ref[...] = jnp.transpose(e_ref[...], (1, 0))

    scores = lax.dot_general(
        x_ref[...].astype(jnp.bfloat16), et_ref[...].astype(jnp.bfloat16),
        (((1,), (0,)), ((), ())),
        preferred_element_type=jnp.float32,
    )
    d = (x2_ref[...][:, None] - 2.0 * scores) + e2_ref[...]
    m = jnp.min(d, axis=1, keepdims=True)
    col = lax.broadcasted_iota(jnp.int32, d.shape, 1)
    idx = jnp.min(jnp.where(d == m, col, K), axis=1)
    idx_ref[...] = idx.astype(jnp.int32)


def _argmin_indices(x_flat, x2, e2, embedding, off, ch):
    blk0 = off // TB
    out = pl.pallas_call(
        _argmin_body,
        grid=(ch // TB,),
        in_specs=[
            pl.BlockSpec((K, D), lambda i: (0, 0)),
            pl.BlockSpec((TB, D), lambda i, b=blk0: (b + i, 0)),
            pl.BlockSpec((TB,), lambda i, b=blk0: (b + i,)),
            pl.BlockSpec((1, K), lambda i: (0, 0)),
        ],
        out_specs=pl.BlockSpec((TB,), lambda i: (i,)),
        out_shape=jax.ShapeDtypeStruct((ch,), jnp.int32),
        scratch_shapes=[
            pltpu.VMEM((D, K), jnp.float32),
        ],
    )(embedding, x_flat, x2, e2)
    return out


def _make_gather(ch):
    info = plsc.get_sparse_core_info()
    nc, ns = info.num_cores, info.num_subcores
    nw = nc * ns
    b_per_w = ch // nw
    half = b_per_w // 2
    mesh = plsc.VectorSubcoreMesh(core_axis_name="c", subcore_axis_name="s")

    @functools.partial(
        pl.kernel,
        mesh=mesh,
        out_type=jax.ShapeDtypeStruct((ch, D), jnp.float32),
        scratch_types=[
            pltpu.VMEM((b_per_w,), jnp.int32),
            pltpu.VMEM((half, D), jnp.float32),
            pltpu.VMEM((half, D), jnp.float32),
            pltpu.SemaphoreType.DMA,
            pltpu.SemaphoreType.DMA,
            pltpu.SemaphoreType.DMA,
            pltpu.SemaphoreType.DMA,
        ],
    )
    def gather(table_hbm, idx_hbm, out_hbm, idx_v, buf0, buf1, sg0, sg1, sw0, sw1):
        wid = lax.axis_index("s") * nc + lax.axis_index("c")
        base = wid * b_per_w
        pltpu.sync_copy(idx_hbm.at[pl.ds(base, b_per_w)], idx_v)
        g0 = pltpu.async_copy(table_hbm.at[idx_v.at[pl.ds(0, half)]], buf0, sg0)
        g1 = pltpu.async_copy(table_hbm.at[idx_v.at[pl.ds(half, half)]], buf1, sg1)
        g0.wait()
        w0 = pltpu.async_copy(buf0, out_hbm.at[pl.ds(base, half)], sw0)
        g1.wait()
        w1 = pltpu.async_copy(buf1, out_hbm.at[pl.ds(base + half, half)], sw1)
        w0.wait()
        w1.wait()

    return gather


def kernel(x, embedding):
    B, T, _ = x.shape
    x_flat = x.reshape(B * T, D)
    x2 = jnp.sum(x_flat ** 2, axis=1)
    e2 = jnp.sum(embedding ** 2, axis=1)[None, :]
    chunks = []
    for off, ch in zip(OFFS, CHUNKS):
        idx_c = _argmin_indices(x_flat, x2, e2, embedding, off, ch)
        chunks.append(_make_gather(ch)(embedding, idx_c))
    quantized = jnp.concatenate(chunks, axis=0)
    return quantized.reshape(B, T, D)


# speedup vs baseline: 1.1921x; 1.1327x over previous
import functools

import jax
import jax.numpy as jnp
from jax import lax
from jax.experimental import pallas as pl
from jax.experimental.pallas import tpu as pltpu
from jax.experimental.pallas import tpu_sc as plsc

N_TOKENS = 4608
D = 768
K = 1024
TB = 512


def _argmin_body(e_ref, x_ref, x2_ref, e2_ref, idx_ref, et_ref):
    @pl.when(pl.program_id(0) == 0)
    def _():
        et_ref[...] = jnp.transpose(e_ref[...], (1, 0))

    scores = lax.dot_general(
        x_ref[...].astype(jnp.bfloat16), et_ref[...].astype(jnp.bfloat16),
        (((1,), (0,)), ((), ())),
        preferred_element_type=jnp.float32,
    )
    d = (x2_ref[...][:, None] - 2.0 * scores) + e2_ref[...]
    m = jnp.min(d, axis=1, keepdims=True)
    col = lax.broadcasted_iota(jnp.int32, d.shape, 1)
    idx = jnp.min(jnp.where(d == m, col, K), axis=1)
    idx_ref[...] = idx.astype(jnp.int32)


def _argmin_indices(x_flat, x2, e2, embedding):
    out = pl.pallas_call(
        _argmin_body,
        grid=(N_TOKENS // TB,),
        in_specs=[
            pl.BlockSpec((K, D), lambda i: (0, 0)),
            pl.BlockSpec((TB, D), lambda i: (i, 0)),
            pl.BlockSpec((TB,), lambda i: (i,)),
            pl.BlockSpec((1, K), lambda i: (0, 0)),
        ],
        out_specs=pl.BlockSpec((TB,), lambda i: (i,)),
        out_shape=jax.ShapeDtypeStruct((N_TOKENS,), jnp.int32),
        scratch_shapes=[
            pltpu.VMEM((D, K), jnp.float32),
        ],
    )(embedding, x_flat, x2, e2)
    return out


def _make_gather():
    info = plsc.get_sparse_core_info()
    nc, ns = info.num_cores, info.num_subcores
    nw = nc * ns
    b_per_w = N_TOKENS // nw
    half = b_per_w // 2
    mesh = plsc.VectorSubcoreMesh(core_axis_name="c", subcore_axis_name="s")

    @functools.partial(
        pl.kernel,
        mesh=mesh,
        out_type=jax.ShapeDtypeStruct((N_TOKENS, D), jnp.float32),
        scratch_types=[
            pltpu.VMEM((b_per_w,), jnp.int32),
            pltpu.VMEM((half, D), jnp.float32),
            pltpu.VMEM((half, D), jnp.float32),
            pltpu.SemaphoreType.DMA,
            pltpu.SemaphoreType.DMA,
            pltpu.SemaphoreType.DMA,
            pltpu.SemaphoreType.DMA,
        ],
    )
    def gather(table_hbm, idx_hbm, out_hbm, idx_v, buf0, buf1, sg0, sg1, sw0, sw1):
        wid = lax.axis_index("s") * nc + lax.axis_index("c")
        base = wid * b_per_w
        pltpu.sync_copy(idx_hbm.at[pl.ds(base, b_per_w)], idx_v)
        g0 = pltpu.async_copy(table_hbm.at[idx_v.at[pl.ds(0, half)]], buf0, sg0)
        g1 = pltpu.async_copy(table_hbm.at[idx_v.at[pl.ds(half, half)]], buf1, sg1)
        g0.wait()
        w0 = pltpu.async_copy(buf0, out_hbm.at[pl.ds(base, half)], sw0)
        g1.wait()
        w1 = pltpu.async_copy(buf1, out_hbm.at[pl.ds(base + half, half)], sw1)
        w0.wait()
        w1.wait()

    return gather


def kernel(x, embedding):
    B, T, _ = x.shape
    x_flat = x.reshape(B * T, D)
    x2 = jnp.sum(x_flat ** 2, axis=1)
    e2 = jnp.sum(embedding ** 2, axis=1)[None, :]
    idx = _argmin_indices(x_flat, x2, e2, embedding)
    quantized = _make_gather()(embedding, idx)
    return quantized.reshape(B, T, D)


# speedup vs baseline: 1.1988x; 1.0056x over previous
import functools

import jax
import jax.numpy as jnp
from jax import lax
from jax.experimental import pallas as pl
from jax.experimental.pallas import tpu as pltpu
from jax.experimental.pallas import tpu_sc as plsc

N_TOKENS = 4608
D = 768
K = 1024
TB = 512


def _argmin_body(e_ref, x_ref, x2_ref, e2_ref, idx_ref, et_ref):
    @pl.when(pl.program_id(0) == 0)
    def _():
        et_ref[...] = jnp.transpose(e_ref[...], (1, 0))

    scores = lax.dot_general(
        x_ref[...].astype(jnp.bfloat16), et_ref[...].astype(jnp.bfloat16),
        (((1,), (0,)), ((), ())),
        preferred_element_type=jnp.float32,
    )
    d = (x2_ref[...][:, None] - 2.0 * scores) + e2_ref[...]
    m = jnp.min(d, axis=1, keepdims=True)
    col = lax.broadcasted_iota(jnp.int32, d.shape, 1)
    idx = jnp.min(jnp.where(d == m, col, K), axis=1)
    idx_ref[...] = idx.astype(jnp.int32)


def _argmin_indices(x_flat, x2, e2, embedding):
    out = pl.pallas_call(
        _argmin_body,
        grid=(N_TOKENS // TB,),
        in_specs=[
            pl.BlockSpec((K, D), lambda i: (0, 0)),
            pl.BlockSpec((TB, D), lambda i: (i, 0)),
            pl.BlockSpec((TB,), lambda i: (i,)),
            pl.BlockSpec((1, K), lambda i: (0, 0)),
        ],
        out_specs=pl.BlockSpec((TB,), lambda i: (i,)),
        out_shape=jax.ShapeDtypeStruct((N_TOKENS,), jnp.int32),
        scratch_shapes=[
            pltpu.VMEM((D, K), jnp.float32),
        ],
    )(embedding, x_flat, x2, e2)
    return out


def _make_gather():
    info = plsc.get_sparse_core_info()
    nc, ns = info.num_cores, info.num_subcores
    nw = nc * ns
    b_per_w = N_TOKENS // nw
    half = b_per_w // 2
    mesh = plsc.VectorSubcoreMesh(core_axis_name="c", subcore_axis_name="s")

    q = b_per_w // 3

    @functools.partial(
        pl.kernel,
        mesh=mesh,
        out_type=jax.ShapeDtypeStruct((N_TOKENS, D), jnp.float32),
        scratch_types=[
            pltpu.VMEM((b_per_w,), jnp.int32),
            pltpu.VMEM((q, D), jnp.float32),
            pltpu.VMEM((q, D), jnp.float32),
            pltpu.VMEM((q, D), jnp.float32),
            pltpu.SemaphoreType.DMA,
            pltpu.SemaphoreType.DMA,
            pltpu.SemaphoreType.DMA,
            pltpu.SemaphoreType.DMA,
            pltpu.SemaphoreType.DMA,
            pltpu.SemaphoreType.DMA,
        ],
    )
    def gather(table_hbm, idx_hbm, out_hbm, idx_v, b0, b1, b2,
               g0s, g1s, g2s, w0s, w1s, w2s):
        wid = lax.axis_index("s") * nc + lax.axis_index("c")
        base = wid * b_per_w
        pltpu.sync_copy(idx_hbm.at[pl.ds(base, b_per_w)], idx_v)
        bufs = (b0, b1, b2)
        gsems = (g0s, g1s, g2s)
        wsems = (w0s, w1s, w2s)
        gops = []
        for j in range(3):
            gops.append(pltpu.async_copy(
                table_hbm.at[idx_v.at[pl.ds(j * q, q)]], bufs[j], gsems[j]))
        wops = []
        for j in range(3):
            gops[j].wait()
            wops.append(pltpu.async_copy(
                bufs[j], out_hbm.at[pl.ds(base + j * q, q)], wsems[j]))
        for j in range(3):
            wops[j].wait()

    return gather


def kernel(x, embedding):
    B, T, _ = x.shape
    x_flat = x.reshape(B * T, D)
    x2 = jnp.sum(x_flat ** 2, axis=1)
    e2 = jnp.sum(embedding ** 2, axis=1)[None, :]
    idx = _argmin_indices(x_flat, x2, e2, embedding)
    quantized = _make_gather()(embedding, idx)
    return quantized.reshape(B, T, D)
